# initial kernel scaffold (unmeasured)
import jax
import jax.numpy as jnp
from jax import lax
from jax.experimental import pallas as pl
from jax.experimental.pallas import tpu as pltpu


def kernel(
    x,
):
    def body(*refs):
        pass

    out_shape = jax.ShapeDtypeStruct(..., jnp.float32)
    return pl.pallas_call(body, out_shape=out_shape)(...)


# baseline (device time: 10573 ns/iter reference)
import jax
import jax.numpy as jnp
from jax import lax
from jax.experimental import pallas as pl
from jax.experimental.pallas import tpu as pltpu

N_DEV = 4


def kernel(x):
    _, m, n = x.shape

    def body(x_ref, o_ref, acc_ref, recv_ref, send_sems, recv_sems):
        my = lax.axis_index("i")
        p1 = my ^ 1
        p2 = 3 - my

        barrier_sem = pltpu.get_barrier_semaphore()
        for p in (p1, p2):
            pl.semaphore_signal(
                barrier_sem, inc=1,
                device_id=(p,), device_id_type=pl.DeviceIdType.MESH,
            )
        pl.semaphore_wait(barrier_sem, 2)

        acc_ref[...] = x_ref[0].astype(jnp.bfloat16)

        rdma1 = pltpu.make_async_remote_copy(
            src_ref=acc_ref,
            dst_ref=recv_ref.at[0],
            send_sem=send_sems.at[0],
            recv_sem=recv_sems.at[0],
            device_id=(p1,),
            device_id_type=pl.DeviceIdType.MESH,
        )
        rdma1.start()
        rdma1.wait()
        acc_ref[...] = acc_ref[...] + recv_ref[0]

        rdma2 = pltpu.make_async_remote_copy(
            src_ref=acc_ref,
            dst_ref=recv_ref.at[1],
            send_sem=send_sems.at[1],
            recv_sem=recv_sems.at[1],
            device_id=(p2,),
            device_id_type=pl.DeviceIdType.MESH,
        )
        rdma2.start()
        rdma2.wait()
        o_ref[...] = (acc_ref[...] + recv_ref[1]).astype(jnp.float32)

    return pl.pallas_call(
        body,
        out_shape=jax.ShapeDtypeStruct((m, n), jnp.float32),
        in_specs=[pl.BlockSpec(memory_space=pltpu.VMEM)],
        out_specs=pl.BlockSpec(memory_space=pltpu.VMEM),
        scratch_shapes=[
            pltpu.VMEM((m, n), jnp.bfloat16),
            pltpu.VMEM((2, m, n), jnp.bfloat16),
            pltpu.SemaphoreType.DMA((2,)),
            pltpu.SemaphoreType.DMA((2,)),
        ],
        compiler_params=pltpu.CompilerParams(collective_id=0),
    )(x)


# device time: 9202 ns/iter; 1.1490x vs baseline; 1.1490x over previous
import jax
import jax.numpy as jnp
from jax import lax
from jax.experimental import pallas as pl
from jax.experimental.pallas import tpu as pltpu

N_DEV = 4


def kernel(x):
    _, m, n = x.shape

    half = m // 2

    def body(x_ref, o_ref, acc_ref, recv_ref, send_sems, recv_sems):
        my = lax.axis_index("i")
        p1 = my ^ 1
        p2 = 3 - my

        barrier_sem = pltpu.get_barrier_semaphore()
        for p in (p1, p2):
            pl.semaphore_signal(
                barrier_sem, inc=1,
                device_id=(p,), device_id_type=pl.DeviceIdType.MESH,
            )
        pl.semaphore_wait(barrier_sem, 2)

        acc_ref[...] = x_ref[0].astype(jnp.bfloat16)

        a = pl.ds(0, half)
        b = pl.ds(half, half)

        def exchange(stage, sl, partner):
            return pltpu.make_async_remote_copy(
                src_ref=acc_ref.at[sl],
                dst_ref=recv_ref.at[stage, 0 if sl is a else 1],
                send_sem=send_sems.at[stage, 0 if sl is a else 1],
                recv_sem=recv_sems.at[stage, 0 if sl is a else 1],
                device_id=(partner,),
                device_id_type=pl.DeviceIdType.MESH,
            )

        r1a = exchange(0, a, p1)
        r1b = exchange(0, b, p2)
        r1a.start()
        r1b.start()

        r1a.wait()
        acc_ref[a] = acc_ref[a] + recv_ref[0, 0]
        r2a = exchange(1, a, p2)
        r2a.start()

        r1b.wait()
        acc_ref[b] = acc_ref[b] + recv_ref[0, 1]
        r2b = exchange(1, b, p1)
        r2b.start()

        r2a.wait()
        o_ref[a] = (acc_ref[a] + recv_ref[1, 0]).astype(jnp.float32)
        r2b.wait()
        o_ref[b] = (acc_ref[b] + recv_ref[1, 1]).astype(jnp.float32)

    return pl.pallas_call(
        body,
        out_shape=jax.ShapeDtypeStruct((m, n), jnp.float32),
        in_specs=[pl.BlockSpec(memory_space=pltpu.VMEM)],
        out_specs=pl.BlockSpec(memory_space=pltpu.VMEM),
        scratch_shapes=[
            pltpu.VMEM((m, n), jnp.bfloat16),
            pltpu.VMEM((2, 2, half, n), jnp.bfloat16),
            pltpu.SemaphoreType.DMA((2, 2)),
            pltpu.SemaphoreType.DMA((2, 2)),
        ],
        compiler_params=pltpu.CompilerParams(collective_id=0),
    )(x)


# device time: 9180 ns/iter; 1.1517x vs baseline; 1.0024x over previous
import jax
import jax.numpy as jnp
from jax import lax
from jax.experimental import pallas as pl
from jax.experimental.pallas import tpu as pltpu

N_DEV = 4


def kernel(x):
    _, m, n = x.shape

    half = m // 2

    def body(x_ref, o_ref, acc_ref, sum1_ref, recv_ref, send_sems, recv_sems):
        my = lax.axis_index("i")
        p1 = my ^ 1
        p2 = 3 - my

        barrier_sem = pltpu.get_barrier_semaphore()
        for p in (p1, p2):
            pl.semaphore_signal(
                barrier_sem, inc=1,
                device_id=(p,), device_id_type=pl.DeviceIdType.MESH,
            )
        pl.semaphore_wait(barrier_sem, 2)

        acc_ref[...] = x_ref[0].astype(jnp.bfloat16)

        a = pl.ds(0, half)
        b = pl.ds(half, half)

        def exchange(stage, sl, partner):
            return pltpu.make_async_remote_copy(
                src_ref=acc_ref.at[sl],
                dst_ref=recv_ref.at[stage, 0 if sl is a else 1],
                send_sem=send_sems.at[stage, 0 if sl is a else 1],
                recv_sem=recv_sems.at[stage, 0 if sl is a else 1],
                device_id=(partner,),
                device_id_type=pl.DeviceIdType.MESH,
            )

        r1a = exchange(0, a, p1)
        r1b = exchange(0, b, p2)
        r1a.start()
        r1b.start()

        r1a.wait_recv()
        sum1_ref[a] = acc_ref[a] + recv_ref[0, 0]
        r2a = pltpu.make_async_remote_copy(
            src_ref=sum1_ref.at[a],
            dst_ref=recv_ref.at[1, 0],
            send_sem=send_sems.at[1, 0],
            recv_sem=recv_sems.at[1, 0],
            device_id=(p2,),
            device_id_type=pl.DeviceIdType.MESH,
        )
        r2a.start()

        r1b.wait_recv()
        sum1_ref[b] = acc_ref[b] + recv_ref[0, 1]
        r2b = pltpu.make_async_remote_copy(
            src_ref=sum1_ref.at[b],
            dst_ref=recv_ref.at[1, 1],
            send_sem=send_sems.at[1, 1],
            recv_sem=recv_sems.at[1, 1],
            device_id=(p1,),
            device_id_type=pl.DeviceIdType.MESH,
        )
        r2b.start()

        r2a.wait_recv()
        o_ref[a] = sum1_ref[a] + recv_ref[1, 0]
        r2b.wait_recv()
        o_ref[b] = sum1_ref[b] + recv_ref[1, 1]

        r1a.wait_send()
        r1b.wait_send()
        r2a.wait_send()
        r2b.wait_send()

    return pl.pallas_call(
        body,
        out_shape=jax.ShapeDtypeStruct((m, n), jnp.bfloat16),
        in_specs=[pl.BlockSpec(memory_space=pltpu.VMEM)],
        out_specs=pl.BlockSpec(memory_space=pltpu.VMEM),
        scratch_shapes=[
            pltpu.VMEM((m, n), jnp.bfloat16),
            pltpu.VMEM((m, n), jnp.bfloat16),
            pltpu.VMEM((2, 2, half, n), jnp.bfloat16),
            pltpu.SemaphoreType.DMA((2, 2)),
            pltpu.SemaphoreType.DMA((2, 2)),
        ],
        compiler_params=pltpu.CompilerParams(collective_id=0),
    )(x)


# device time: 8881 ns/iter; 1.1905x vs baseline; 1.0337x over previous
import jax
import jax.numpy as jnp
from jax import lax
from jax.experimental import pallas as pl
from jax.experimental.pallas import tpu as pltpu

N_DEV = 4


def kernel(x):
    _, m, n = x.shape

    q = m // 4
    ORDER = (0, 2, 1, 3)

    def body(x_ref, o_ref, acc_ref, sum1_ref, recv1_ref, recv2_ref,
             send_sems1, recv_sems1, send_sems2, recv_sems2):
        my = lax.axis_index("i")
        p1 = my ^ 1
        p2 = 3 - my
        stage1_to = {0: p1, 1: p1, 2: p2, 3: p2}
        stage2_to = {0: p2, 1: p2, 2: p1, 3: p1}

        barrier_sem = pltpu.get_barrier_semaphore()
        for p in (p1, p2):
            pl.semaphore_signal(
                barrier_sem, inc=1,
                device_id=(p,), device_id_type=pl.DeviceIdType.MESH,
            )
        acc_ref[...] = x_ref[0].astype(jnp.bfloat16)
        pl.semaphore_wait(barrier_sem, 2)

        sl = {k: pl.ds(k * q, q) for k in range(4)}

        r1 = {}
        for k in ORDER:
            r1[k] = pltpu.make_async_remote_copy(
                src_ref=acc_ref.at[sl[k]],
                dst_ref=recv1_ref.at[k],
                send_sem=send_sems1.at[k],
                recv_sem=recv_sems1.at[k],
                device_id=(stage1_to[k],),
                device_id_type=pl.DeviceIdType.MESH,
            )
            r1[k].start()

        r2 = {}
        for k in ORDER:
            r1[k].wait_recv()
            sum1_ref[sl[k]] = acc_ref[sl[k]] + recv1_ref[k]
            r2[k] = pltpu.make_async_remote_copy(
                src_ref=sum1_ref.at[sl[k]],
                dst_ref=recv2_ref.at[k],
                send_sem=send_sems2.at[k],
                recv_sem=recv_sems2.at[k],
                device_id=(stage2_to[k],),
                device_id_type=pl.DeviceIdType.MESH,
            )
            r2[k].start()

        for k in ORDER:
            r2[k].wait_recv()
            o_ref[sl[k]] = sum1_ref[sl[k]] + recv2_ref[k]

        for k in ORDER:
            r1[k].wait_send()
            r2[k].wait_send()

    return pl.pallas_call(
        body,
        out_shape=jax.ShapeDtypeStruct((m, n), jnp.bfloat16),
        in_specs=[pl.BlockSpec(memory_space=pltpu.VMEM)],
        out_specs=pl.BlockSpec(memory_space=pltpu.VMEM),
        scratch_shapes=[
            pltpu.VMEM((m, n), jnp.bfloat16),
            pltpu.VMEM((m, n), jnp.bfloat16),
            pltpu.VMEM((4, q, n), jnp.bfloat16),
            pltpu.VMEM((4, q, n), jnp.bfloat16),
            pltpu.SemaphoreType.DMA((4,)),
            pltpu.SemaphoreType.DMA((4,)),
            pltpu.SemaphoreType.DMA((4,)),
            pltpu.SemaphoreType.DMA((4,)),
        ],
        compiler_params=pltpu.CompilerParams(collective_id=0),
    )(x)
